# baseline (speedup 1.0000x reference)
import functools

import jax
import jax.numpy as jnp
from jax import lax
from jax.experimental import pallas as pl
from jax.experimental.pallas import tpu as pltpu
from jax.experimental.pallas import tpu_sc as plsc

N = 10000
E = 320000
NT = 16
CH = 128
SL = 640
N_PAD = NT * SL
NCHUNK = 160
ET = NCHUNK * CH
NV = SL // 16


def _rsqrt16(d):
    i = plsc.bitcast(d, jnp.int32)
    i = jnp.int32(0x5F3759DF) - lax.shift_right_logical(i, jnp.int32(1))
    y = plsc.bitcast(i, jnp.float32)
    for _ in range(3):
        y = y * (jnp.float32(1.5) - jnp.float32(0.5) * d * y * y)
    return y


def _sc_body(ei_h, pad_h, p_h, par_h,
             h_o,
             src_v, dst_v, dinv_f, q_f, p2_f,
             mb0, mb1, buf_a, buf_b, buf_c, par_v,
             dinv_s, h10_s, h11_s, h2_s, q_s,
             sem_in, sem0, sem1):
    tid = lax.axis_index("s")
    base = tid * SL
    sl = pl.ds(base, SL)
    f32 = jnp.float32
    i32 = jnp.int32
    iota16 = lax.iota(i32, 16)
    c0 = jnp.zeros((16,), i32)
    c1 = jnp.ones((16,), i32)
    zero16 = jnp.zeros((16,), f32)

    cp_p = pltpu.async_copy(p_h, p2_f, sem_in)
    pltpu.sync_copy(par_h, par_v)

    ETAIL = E - (NT - 1) * ET

    @pl.when(tid < NT - 1)
    def _stage_full():
        pltpu.sync_copy(ei_h.at[pl.ds(0, 1), pl.ds(tid * ET, ET)], src_v)
        pltpu.sync_copy(ei_h.at[pl.ds(1, 1), pl.ds(tid * ET, ET)], dst_v)

    @pl.when(tid == NT - 1)
    def _stage_tail():
        pltpu.sync_copy(ei_h.at[pl.ds(0, 1), pl.ds((NT - 1) * ET, ETAIL)],
                        src_v.at[pl.ds(0, 1), pl.ds(0, ETAIL)])
        pltpu.sync_copy(ei_h.at[pl.ds(1, 1), pl.ds((NT - 1) * ET, ETAIL)],
                        dst_v.at[pl.ds(0, 1), pl.ds(0, ETAIL)])
        pltpu.sync_copy(pad_h, src_v.at[0, pl.ds(ETAIL, ET - ETAIL)])
        pltpu.sync_copy(pad_h, dst_v.at[0, pl.ds(ETAIL, ET - ETAIL)])

    for v in range(NV):
        buf_a[pl.ds(16 * v, 16)] = zero16
        buf_b[pl.ds(16 * v, 16)] = jnp.ones((16,), f32)
    pltpu.sync_copy(buf_b, dinv_s.at[sl])
    pltpu.sync_copy(buf_a, h10_s.at[sl])
    pltpu.sync_copy(buf_a, h11_s.at[sl])
    pltpu.sync_copy(buf_a, h2_s.at[sl])
    cp_p.wait()
    plsc.subcore_barrier()

    ones_ch = buf_b.at[pl.ds(0, CH)]

    def _deg_issue(j):
        pltpu.async_copy(ones_ch, dinv_s.at[dst_v.at[0, pl.ds(j * CH, CH)]],
                         sem0, add=True)

    def _deg_drain(j, c):
        pltpu.make_async_copy(ones_ch,
                              dinv_s.at[dst_v.at[0, pl.ds(j * CH, CH)]],
                              sem0).wait()
        return c

    with jax.named_scope("ph1_deg"):
        plsc.parallel_loop(0, NCHUNK, 1, unroll=4)(_deg_issue)
        lax.fori_loop(0, NCHUNK, _deg_drain, 0)
    plsc.subcore_barrier()

    pltpu.sync_copy(dinv_s.at[sl], buf_a)
    for v in range(NV):
        w = pl.ds(16 * v, 16)
        buf_a[w] = _rsqrt16(buf_a[w])
    pltpu.sync_copy(buf_a, dinv_s.at[sl])
    plsc.subcore_barrier()
    pltpu.sync_copy(dinv_s, dinv_f)

    def _c1(j):
        for k in range(CH // 16):
            w = pl.ds(16 * k, 16)
            e = pl.ds(j * CH + 16 * k, 16)
            s = src_v[0, e]
            t = dst_v[0, e]
            nrm = plsc.load_gather(dinv_f, [s]) * plsc.load_gather(dinv_f, [t])
            e0 = lax.shift_left(s, jnp.int32(1))
            mb0[j, w] = plsc.load_gather(
                p2_f, [lax.shift_right_logical(e0, jnp.int32(7)),
                       lax.bitwise_and(e0, jnp.int32(127))]) * nrm
            e1 = e0 + jnp.int32(1)
            mb1[j, w] = plsc.load_gather(
                p2_f, [lax.shift_right_logical(e1, jnp.int32(7)),
                       lax.bitwise_and(e1, jnp.int32(127))]) * nrm
        idx = dst_v.at[0, pl.ds(j * CH, CH)]
        pltpu.async_copy(mb0.at[j], h10_s.at[idx], sem0, add=True)
        pltpu.async_copy(mb1.at[j], h11_s.at[idx], sem1, add=True)

    def _c1_drain(j, c):
        idx = dst_v.at[0, pl.ds(j * CH, CH)]
        pltpu.make_async_copy(mb0.at[j], h10_s.at[idx], sem0).wait()
        pltpu.make_async_copy(mb1.at[j], h11_s.at[idx], sem1).wait()
        return c

    with jax.named_scope("ph3_conv1"):
        plsc.parallel_loop(0, NCHUNK, 1, unroll=4)(_c1)
        lax.fori_loop(0, NCHUNK, _c1_drain, 0)
    plsc.subcore_barrier()

    pltpu.sync_copy(h10_s.at[sl], buf_a)
    pltpu.sync_copy(h11_s.at[sl], buf_b)

    def _splat(k):
        return plsc.load_gather(par_v, [jnp.full((16,), k, i32)])

    w20 = _splat(0)
    w21 = _splat(1)
    b10 = _splat(2)
    b11 = _splat(3)
    b2v = _splat(4)
    for v in range(NV):
        w = pl.ds(16 * v, 16)
        g = pl.ds(base + 16 * v, 16)
        i16 = iota16 + jnp.int32(16 * v)
        gi = i16 + base
        dv = dinv_f[g]
        sn = dv * dv
        e0 = lax.shift_left(gi, jnp.int32(1))
        e1 = e0 + jnp.int32(1)
        h0 = buf_a[w] + plsc.load_gather(
            p2_f, [lax.shift_right_logical(e0, jnp.int32(7)),
                   lax.bitwise_and(e0, jnp.int32(127))]) * sn + b10
        h1 = buf_b[w] + plsc.load_gather(
            p2_f, [lax.shift_right_logical(e1, jnp.int32(7)),
                   lax.bitwise_and(e1, jnp.int32(127))]) * sn + b11
        buf_a[w] = h0
        buf_b[w] = h1
        buf_c[w] = h0 * w20 + h1 * w21
    pltpu.sync_copy(buf_a, h_o.at[pl.ds(base, SL)])
    pltpu.sync_copy(buf_b, h_o.at[pl.ds(N_PAD + base, SL)])
    pltpu.sync_copy(buf_c, q_s.at[sl])
    plsc.subcore_barrier()
    pltpu.sync_copy(q_s, q_f)

    def _c2(j):
        for k in range(CH // 16):
            w = pl.ds(16 * k, 16)
            e = pl.ds(j * CH + 16 * k, 16)
            s = src_v[0, e]
            t = dst_v[0, e]
            nrm = plsc.load_gather(dinv_f, [s]) * plsc.load_gather(dinv_f, [t])
            mb0[j, w] = plsc.load_gather(q_f, [s]) * nrm
        pltpu.async_copy(mb0.at[j], h2_s.at[dst_v.at[0, pl.ds(j * CH, CH)]],
                         sem0, add=True)

    def _c2_drain(j, c):
        pltpu.make_async_copy(mb0.at[j],
                              h2_s.at[dst_v.at[0, pl.ds(j * CH, CH)]],
                              sem0).wait()
        return c

    with jax.named_scope("ph5_conv2"):
        plsc.parallel_loop(0, NCHUNK, 1, unroll=4)(_c2)
        lax.fori_loop(0, NCHUNK, _c2_drain, 0)
    plsc.subcore_barrier()

    pltpu.sync_copy(h2_s.at[sl], buf_a)
    for v in range(NV):
        w = pl.ds(16 * v, 16)
        g = pl.ds(base + 16 * v, 16)
        dv = dinv_f[g]
        buf_a[w] = buf_a[w] + q_f[g] * dv * dv + b2v
    pltpu.sync_copy(buf_a, h_o.at[pl.ds(2 * N_PAD + base, SL)])


_sc_edges = functools.partial(
    pl.kernel,
    out_type=jax.ShapeDtypeStruct((3 * N_PAD,), jnp.float32),
    mesh=plsc.VectorSubcoreMesh(core_axis_name="c", subcore_axis_name="s",
                                num_cores=1, num_subcores=NT),
    compiler_params=pltpu.CompilerParams(needs_layout_passes=False),
    scratch_types=[
        pltpu.VMEM((1, ET), jnp.int32),
        pltpu.VMEM((1, ET), jnp.int32),
        pltpu.VMEM((N_PAD,), jnp.float32),
        pltpu.VMEM((N_PAD,), jnp.float32),
        pltpu.VMEM((2 * N_PAD // CH, CH), jnp.float32),
        pltpu.VMEM((NCHUNK, CH), jnp.float32),
        pltpu.VMEM((NCHUNK, CH), jnp.float32),
        pltpu.VMEM((SL,), jnp.float32),
        pltpu.VMEM((SL,), jnp.float32),
        pltpu.VMEM((SL,), jnp.float32),
        pltpu.VMEM((16,), jnp.float32),
        pltpu.VMEM_SHARED((N_PAD,), jnp.float32),
        pltpu.VMEM_SHARED((N_PAD,), jnp.float32),
        pltpu.VMEM_SHARED((N_PAD,), jnp.float32),
        pltpu.VMEM_SHARED((N_PAD,), jnp.float32),
        pltpu.VMEM_SHARED((N_PAD,), jnp.float32),
        pltpu.SemaphoreType.DMA,
        pltpu.SemaphoreType.DMA,
        pltpu.SemaphoreType.DMA,
    ],
)(_sc_body)


def _mm_body(x_ref, w_ref, o_ref):
    o_ref[pl.ds(0, N), :] = jnp.dot(x_ref[...], w_ref[...],
                                    preferred_element_type=jnp.float32)
    o_ref[pl.ds(N, N_PAD - N), :] = jnp.zeros((N_PAD - N, 2), jnp.float32)


def _tc_matmul(x, w):
    return pl.pallas_call(
        _mm_body,
        out_shape=jax.ShapeDtypeStruct((N_PAD, 2), jnp.float32),
    )(x, w)


def _head_body(xs, cw0, cw1, cw2, cb, wout, bout, wh, bh, o_ref):
    td = (xs[0] * cw0[...] + xs[1] * cw1[...] + xs[2] * cw2[...]
          + cb[...])
    m = jnp.dot(wout[...], wh[...], preferred_element_type=jnp.float32)
    const = jnp.dot(bout[...], wh[...], preferred_element_type=jnp.float32)
    logits = jnp.dot(td, m, preferred_element_type=jnp.float32) + const + bh[...]
    mx = jnp.max(logits, axis=1, keepdims=True)
    ls = logits - mx
    o_ref[...] = ls - jnp.log(jnp.sum(jnp.exp(ls), axis=1, keepdims=True))


def _tc_head(xs, cw0, cw1, cw2, cb, wout, bout, wh, bh):
    return pl.pallas_call(
        _head_body,
        out_shape=jax.ShapeDtypeStruct((xs.shape[1], wh.shape[1]),
                                       jnp.float32),
    )(xs, cw0, cw1, cw2, cb, wout, bout, wh, bh)


def kernel(x, edge_attr, W1, b1, W2, b2, convW, convB, Wout, bout, Wh, bh,
           edge_index, batch):
    p = _tc_matmul(x, W1)

    pad = ET * NT - E
    pad_idx = (N + jnp.arange(pad, dtype=jnp.int32) % (N_PAD - N)).astype(jnp.int32)

    par = jnp.concatenate([W2.reshape(2), b1, b2,
                           jnp.zeros(11, jnp.float32)])
    h = _sc_edges(edge_index, pad_idx, p.reshape(2 * N_PAD // CH, CH), par)

    xs = h.reshape(3, N_PAD)[:, :N].reshape(3, N // 50, 50)
    cw0 = convW[:, 0].reshape(1, 50)
    cw1 = convW[:, 1].reshape(1, 50)
    cw2 = convW[:, 2].reshape(1, 50)
    cb = convB.reshape(1, 50)
    return _tc_head(xs, cw0, cw1, cw2, cb, Wout,
                    bout.reshape(1, 128), Wh, bh.reshape(1, 16))


# speedup vs baseline: 1.0141x; 1.0141x over previous
import functools

import jax
import jax.numpy as jnp
from jax import lax
from jax.experimental import pallas as pl
from jax.experimental.pallas import tpu as pltpu
from jax.experimental.pallas import tpu_sc as plsc

N = 10000
E = 320000
NT = 16
CH = 128
SL = 640
N_PAD = NT * SL
NCHUNK = 160
ET = NCHUNK * CH
NV = SL // 16


def _rsqrt16(d):
    i = plsc.bitcast(d, jnp.int32)
    i = jnp.int32(0x5F3759DF) - lax.shift_right_logical(i, jnp.int32(1))
    y = plsc.bitcast(i, jnp.float32)
    for _ in range(3):
        y = y * (jnp.float32(1.5) - jnp.float32(0.5) * d * y * y)
    return y


def _sc_body(ei_h, pad_h, p_h, par_h,
             h_o,
             src_v, dst_v, dinv_f, q_f, p2_f,
             mb0, mb1, buf_a, buf_b, buf_c, par_v,
             dinv_s, h10_s, h11_s, h2_s, q_s,
             sem_in, sem0, sem1):
    tid = lax.axis_index("s")
    base = tid * SL
    sl = pl.ds(base, SL)
    f32 = jnp.float32
    i32 = jnp.int32
    iota16 = lax.iota(i32, 16)
    c0 = jnp.zeros((16,), i32)
    c1 = jnp.ones((16,), i32)
    zero16 = jnp.zeros((16,), f32)

    cp_p = pltpu.async_copy(p_h, p2_f, sem_in)
    pltpu.sync_copy(par_h, par_v)

    ETAIL = E - (NT - 1) * ET

    @pl.when(tid < NT - 1)
    def _stage_full():
        pltpu.sync_copy(ei_h.at[pl.ds(0, 1), pl.ds(tid * ET, ET)], src_v)
        pltpu.sync_copy(ei_h.at[pl.ds(1, 1), pl.ds(tid * ET, ET)], dst_v)

    @pl.when(tid == NT - 1)
    def _stage_tail():
        pltpu.sync_copy(ei_h.at[pl.ds(0, 1), pl.ds((NT - 1) * ET, ETAIL)],
                        src_v.at[pl.ds(0, 1), pl.ds(0, ETAIL)])
        pltpu.sync_copy(ei_h.at[pl.ds(1, 1), pl.ds((NT - 1) * ET, ETAIL)],
                        dst_v.at[pl.ds(0, 1), pl.ds(0, ETAIL)])
        pltpu.sync_copy(pad_h, src_v.at[0, pl.ds(ETAIL, ET - ETAIL)])
        pltpu.sync_copy(pad_h, dst_v.at[0, pl.ds(ETAIL, ET - ETAIL)])

    for v in range(NV):
        buf_a[pl.ds(16 * v, 16)] = zero16
        buf_b[pl.ds(16 * v, 16)] = jnp.ones((16,), f32)
    pltpu.sync_copy(buf_b, dinv_s.at[sl])
    pltpu.sync_copy(buf_a, h10_s.at[sl])
    pltpu.sync_copy(buf_a, h11_s.at[sl])
    pltpu.sync_copy(buf_a, h2_s.at[sl])
    cp_p.wait()
    plsc.subcore_barrier()

    ones_ch = buf_b.at[pl.ds(0, CH)]

    def _deg_issue(j):
        pltpu.async_copy(ones_ch, dinv_s.at[dst_v.at[0, pl.ds(j * CH, CH)]],
                         sem0, add=True)

    def _deg_drain(j, c):
        pltpu.make_async_copy(ones_ch,
                              dinv_s.at[dst_v.at[0, pl.ds(j * CH, CH)]],
                              sem0).wait()
        return c

    with jax.named_scope("ph1_deg"):
        plsc.parallel_loop(0, NCHUNK, 1, unroll=2)(_deg_issue)
        lax.fori_loop(0, NCHUNK, _deg_drain, 0)
    plsc.subcore_barrier()

    pltpu.sync_copy(dinv_s.at[sl], buf_a)
    for v in range(NV):
        w = pl.ds(16 * v, 16)
        buf_a[w] = _rsqrt16(buf_a[w])
    pltpu.sync_copy(buf_a, dinv_s.at[sl])
    plsc.subcore_barrier()
    pltpu.sync_copy(dinv_s, dinv_f)

    def _c1(j):
        for k in range(CH // 16):
            w = pl.ds(16 * k, 16)
            e = pl.ds(j * CH + 16 * k, 16)
            s = src_v[0, e]
            t = dst_v[0, e]
            nrm = plsc.load_gather(dinv_f, [s]) * plsc.load_gather(dinv_f, [t])
            e0 = lax.shift_left(s, jnp.int32(1))
            mb0[j, w] = plsc.load_gather(
                p2_f, [lax.shift_right_logical(e0, jnp.int32(7)),
                       lax.bitwise_and(e0, jnp.int32(127))]) * nrm
            e1 = e0 + jnp.int32(1)
            mb1[j, w] = plsc.load_gather(
                p2_f, [lax.shift_right_logical(e1, jnp.int32(7)),
                       lax.bitwise_and(e1, jnp.int32(127))]) * nrm
        idx = dst_v.at[0, pl.ds(j * CH, CH)]
        pltpu.async_copy(mb0.at[j], h10_s.at[idx], sem0, add=True)
        pltpu.async_copy(mb1.at[j], h11_s.at[idx], sem1, add=True)

    def _c1_drain(j, c):
        idx = dst_v.at[0, pl.ds(j * CH, CH)]
        pltpu.make_async_copy(mb0.at[j], h10_s.at[idx], sem0).wait()
        pltpu.make_async_copy(mb1.at[j], h11_s.at[idx], sem1).wait()
        return c

    with jax.named_scope("ph3_conv1"):
        plsc.parallel_loop(0, NCHUNK, 1, unroll=2)(_c1)
        lax.fori_loop(0, NCHUNK, _c1_drain, 0)
    plsc.subcore_barrier()

    pltpu.sync_copy(h10_s.at[sl], buf_a)
    pltpu.sync_copy(h11_s.at[sl], buf_b)

    def _splat(k):
        return plsc.load_gather(par_v, [jnp.full((16,), k, i32)])

    w20 = _splat(0)
    w21 = _splat(1)
    b10 = _splat(2)
    b11 = _splat(3)
    b2v = _splat(4)
    for v in range(NV):
        w = pl.ds(16 * v, 16)
        g = pl.ds(base + 16 * v, 16)
        i16 = iota16 + jnp.int32(16 * v)
        gi = i16 + base
        dv = dinv_f[g]
        sn = dv * dv
        e0 = lax.shift_left(gi, jnp.int32(1))
        e1 = e0 + jnp.int32(1)
        h0 = buf_a[w] + plsc.load_gather(
            p2_f, [lax.shift_right_logical(e0, jnp.int32(7)),
                   lax.bitwise_and(e0, jnp.int32(127))]) * sn + b10
        h1 = buf_b[w] + plsc.load_gather(
            p2_f, [lax.shift_right_logical(e1, jnp.int32(7)),
                   lax.bitwise_and(e1, jnp.int32(127))]) * sn + b11
        buf_a[w] = h0
        buf_b[w] = h1
        buf_c[w] = (h0 * w20 + h1 * w21) * dv
    pltpu.sync_copy(buf_a, h_o.at[pl.ds(base, SL)])
    pltpu.sync_copy(buf_b, h_o.at[pl.ds(N_PAD + base, SL)])
    pltpu.sync_copy(buf_c, q_s.at[sl])
    plsc.subcore_barrier()
    pltpu.sync_copy(q_s, q_f)

    def _c2(j):
        for k in range(CH // 16):
            w = pl.ds(16 * k, 16)
            e = pl.ds(j * CH + 16 * k, 16)
            s = src_v[0, e]
            t = dst_v[0, e]
            mb0[j, w] = (plsc.load_gather(q_f, [s])
                         * plsc.load_gather(dinv_f, [t]))
        pltpu.async_copy(mb0.at[j], h2_s.at[dst_v.at[0, pl.ds(j * CH, CH)]],
                         sem0, add=True)

    def _c2_drain(j, c):
        pltpu.make_async_copy(mb0.at[j],
                              h2_s.at[dst_v.at[0, pl.ds(j * CH, CH)]],
                              sem0).wait()
        return c

    with jax.named_scope("ph5_conv2"):
        plsc.parallel_loop(0, NCHUNK, 1, unroll=2)(_c2)
        lax.fori_loop(0, NCHUNK, _c2_drain, 0)
    plsc.subcore_barrier()

    pltpu.sync_copy(h2_s.at[sl], buf_a)
    for v in range(NV):
        w = pl.ds(16 * v, 16)
        g = pl.ds(base + 16 * v, 16)
        dv = dinv_f[g]
        buf_a[w] = buf_a[w] + q_f[g] * dv + b2v
    pltpu.sync_copy(buf_a, h_o.at[pl.ds(2 * N_PAD + base, SL)])


_sc_edges = functools.partial(
    pl.kernel,
    out_type=jax.ShapeDtypeStruct((3 * N_PAD,), jnp.float32),
    mesh=plsc.VectorSubcoreMesh(core_axis_name="c", subcore_axis_name="s",
                                num_cores=1, num_subcores=NT),
    compiler_params=pltpu.CompilerParams(needs_layout_passes=False),
    scratch_types=[
        pltpu.VMEM((1, ET), jnp.int32),
        pltpu.VMEM((1, ET), jnp.int32),
        pltpu.VMEM((N_PAD,), jnp.float32),
        pltpu.VMEM((N_PAD,), jnp.float32),
        pltpu.VMEM((2 * N_PAD // CH, CH), jnp.float32),
        pltpu.VMEM((NCHUNK, CH), jnp.float32),
        pltpu.VMEM((NCHUNK, CH), jnp.float32),
        pltpu.VMEM((SL,), jnp.float32),
        pltpu.VMEM((SL,), jnp.float32),
        pltpu.VMEM((SL,), jnp.float32),
        pltpu.VMEM((16,), jnp.float32),
        pltpu.VMEM_SHARED((N_PAD,), jnp.float32),
        pltpu.VMEM_SHARED((N_PAD,), jnp.float32),
        pltpu.VMEM_SHARED((N_PAD,), jnp.float32),
        pltpu.VMEM_SHARED((N_PAD,), jnp.float32),
        pltpu.VMEM_SHARED((N_PAD,), jnp.float32),
        pltpu.SemaphoreType.DMA,
        pltpu.SemaphoreType.DMA,
        pltpu.SemaphoreType.DMA,
    ],
)(_sc_body)


def _mm_body(x_ref, w_ref, o_ref):
    o_ref[pl.ds(0, N), :] = jnp.dot(x_ref[...], w_ref[...],
                                    preferred_element_type=jnp.float32)
    o_ref[pl.ds(N, N_PAD - N), :] = jnp.zeros((N_PAD - N, 2), jnp.float32)


def _tc_matmul(x, w):
    return pl.pallas_call(
        _mm_body,
        out_shape=jax.ShapeDtypeStruct((N_PAD, 2), jnp.float32),
    )(x, w)


def _head_body(xs, cw0, cw1, cw2, cb, wout, bout, wh, bh, o_ref):
    td = (xs[0] * cw0[...] + xs[1] * cw1[...] + xs[2] * cw2[...]
          + cb[...])
    m = jnp.dot(wout[...], wh[...], preferred_element_type=jnp.float32)
    const = jnp.dot(bout[...], wh[...], preferred_element_type=jnp.float32)
    logits = jnp.dot(td, m, preferred_element_type=jnp.float32) + const + bh[...]
    mx = jnp.max(logits, axis=1, keepdims=True)
    ls = logits - mx
    o_ref[...] = ls - jnp.log(jnp.sum(jnp.exp(ls), axis=1, keepdims=True))


def _tc_head(xs, cw0, cw1, cw2, cb, wout, bout, wh, bh):
    return pl.pallas_call(
        _head_body,
        out_shape=jax.ShapeDtypeStruct((xs.shape[1], wh.shape[1]),
                                       jnp.float32),
    )(xs, cw0, cw1, cw2, cb, wout, bout, wh, bh)


def kernel(x, edge_attr, W1, b1, W2, b2, convW, convB, Wout, bout, Wh, bh,
           edge_index, batch):
    p = _tc_matmul(x, W1)

    pad = ET * NT - E
    pad_idx = (N + jnp.arange(pad, dtype=jnp.int32) % (N_PAD - N)).astype(jnp.int32)

    par = jnp.concatenate([W2.reshape(2), b1, b2,
                           jnp.zeros(11, jnp.float32)])
    h = _sc_edges(edge_index, pad_idx, p.reshape(2 * N_PAD // CH, CH), par)

    xs = h.reshape(3, N_PAD)[:, :N].reshape(3, N // 50, 50)
    cw0 = convW[:, 0].reshape(1, 50)
    cw1 = convW[:, 1].reshape(1, 50)
    cw2 = convW[:, 2].reshape(1, 50)
    cb = convB.reshape(1, 50)
    return _tc_head(xs, cw0, cw1, cw2, cb, Wout,
                    bout.reshape(1, 128), Wh, bh.reshape(1, 16))
